# baseline (speedup 1.0000x reference)
import functools

import jax
import jax.numpy as jnp
from jax import lax
from jax.experimental import pallas as pl
from jax.experimental.pallas import tpu as pltpu
from jax.experimental.pallas import tpu_sc as plsc

_WINDOW = 512
_D = 16384
_OUT_SLICES = (0, 127, 255, 383, 511)
_SRC_ROWS = tuple(s + 1 for s in _OUT_SLICES if s < _WINDOW - 1)
_NROWS = len(_OUT_SLICES)

_NC = 2
_NS = 16
_NW = _NC * _NS
_C = _D // _NW

_mesh = plsc.VectorSubcoreMesh(core_axis_name="c", subcore_axis_name="s")


@functools.partial(
    pl.kernel,
    mesh=_mesh,
    out_type=jax.ShapeDtypeStruct((_NROWS * _D,), jnp.float32),
    scratch_types=[pltpu.SemaphoreType.DMA],
)
def _gather_rows(x_hbm, buf_hbm, out_hbm, sem):
    wid = lax.axis_index("s") * _NC + lax.axis_index("c")
    base = wid * _C
    copies = []
    for j, r in enumerate(_SRC_ROWS):
        copies.append(pltpu.async_copy(
            buf_hbm.at[pl.ds(r * _D + base, _C)],
            out_hbm.at[pl.ds(j * _D + base, _C)],
            sem))
    copies.append(pltpu.async_copy(
        x_hbm.at[pl.ds(base, _C)],
        out_hbm.at[pl.ds((_NROWS - 1) * _D + base, _C)],
        sem))
    for c in copies:
        c.wait()


def kernel(x, buffer):
    flat = _gather_rows(x, buffer.reshape(_WINDOW * _D))
    return flat.reshape(_NROWS, _D)


# speedup vs baseline: 2.0229x; 2.0229x over previous
import functools

import jax
import jax.numpy as jnp
from jax import lax
from jax.experimental import pallas as pl
from jax.experimental.pallas import tpu as pltpu
from jax.experimental.pallas import tpu_sc as plsc

_WINDOW = 512
_D = 16384
_OUT_SLICES = (0, 127, 255, 383, 511)
_SRC_ROWS = tuple(s + 1 for s in _OUT_SLICES if s < _WINDOW - 1)
_NROWS = len(_OUT_SLICES)

_NC = 2
_NS = 16
_NW = _NC * _NS
_C = _D // _NW

_mesh = plsc.VectorSubcoreMesh(core_axis_name="c", subcore_axis_name="s")


@functools.partial(
    pl.kernel,
    mesh=_mesh,
    out_type=jax.ShapeDtypeStruct((_NROWS, _D), jnp.float32),
    scratch_types=[pltpu.SemaphoreType.DMA],
)
def _gather_rows(x_hbm, buf_hbm, out_hbm, sem):
    wid = lax.axis_index("s") * _NC + lax.axis_index("c")
    base = wid * _C
    copies = []
    for j, r in enumerate(_SRC_ROWS):
        copies.append(pltpu.async_copy(
            buf_hbm.at[pl.ds(r, 1), pl.ds(base, _C)],
            out_hbm.at[pl.ds(j, 1), pl.ds(base, _C)],
            sem))
    copies.append(pltpu.async_copy(
        x_hbm.at[pl.ds(0, 1), pl.ds(base, _C)],
        out_hbm.at[pl.ds(_NROWS - 1, 1), pl.ds(base, _C)],
        sem))
    for c in copies:
        c.wait()


def kernel(x, buffer):
    return _gather_rows(x.reshape(1, _D), buffer)
